# baseline (speedup 1.0000x reference)
import jax
import jax.numpy as jnp
from jax import lax
from jax.experimental import pallas as pl
from jax.experimental.pallas import tpu as pltpu
from jax.experimental.pallas import tpu_sc as plsc

TOTAL_NODES = 131072
H = 128
B = 1024

NC = 2
NS = 16
NW = NC * NS
ROWS_PER_TILE = TOTAL_NODES // NW
SCHUNK = 128
LCHUNK = 128
SPL = LCHUNK // SCHUNK
NLOAD = ROWS_PER_TILE // LCHUNK
NBUF = 6
NRING = (NLOAD // NBUF) * NBUF
NIDX = ROWS_PER_TILE // SCHUNK
BALL_PER_TILE = B // NW
ACC_PER_TILE = B // NS


def _sc_body(nodes, bidx, bptr, part_out, cnt_out, ball_out,
             idx_v, rows_v, ones_v, bptr_v, ball_v, acc_v, cntr_v,
             acc_sh, cnt_sh, semg, seml, sems):
  c = lax.axis_index("c")
  s = lax.axis_index("s")
  wid = c * NS + s
  row0 = wid * ROWS_PER_TILE

  def load(j, b, sem):
    return pltpu.async_copy(
        nodes.at[pl.ds(row0 + j * LCHUNK, LCHUNK)], rows_v.at[b], sem)

  pltpu.sync_copy(bptr.at[pl.ds(wid * BALL_PER_TILE, BALL_PER_TILE)], bptr_v)
  ball_d = pltpu.async_copy(nodes.at[bptr_v], ball_v, semg)
  for b in range(NBUF):
    load(b, b, seml.at[b])
  pltpu.sync_copy(bidx.at[wid], idx_v)

  zv = jnp.zeros((16,), jnp.float32)

  @pl.loop(0, ACC_PER_TILE)
  def _(i):
    for k in range(H // 16):
      acc_v[i, pl.ds(k * 16, 16)] = zv

  for k in range(ACC_PER_TILE // 16):
    cntr_v[pl.ds(k * 16, 16)] = zv
  pltpu.sync_copy(acc_v, acc_sh.at[pl.ds(s * ACC_PER_TILE, ACC_PER_TILE)])
  pltpu.sync_copy(cntr_v, cnt_sh.at[pl.ds(s * ACC_PER_TILE, ACC_PER_TILE)])

  for k in range(SCHUNK // 16):
    ones_v[pl.ds(k * 16, 16)] = jnp.ones((16,), jnp.float32)

  plsc.subcore_barrier()

  def fire_scatters(j, b, sem):
    return [
        pltpu.async_copy(rows_v.at[b], acc_sh.at[idx_v.at[j]], sem, add=True),
        pltpu.async_copy(ones_v, cnt_sh.at[idx_v.at[j]], sem, add=True),
    ]

  LAG = 2

  @pl.loop(0, NRING, step=NBUF)
  def _(j):
    ds = {}

    def drain_and_refill(bb):
      for d in ds.pop(bb):
        d.wait()

      @pl.when(j + NBUF + bb < NLOAD)
      def _():
        load(j + NBUF + bb, bb, seml.at[bb])

    for b in range(NBUF):
      pltpu.make_async_copy(
          nodes.at[pl.ds(row0, LCHUNK)], rows_v.at[b], seml.at[b]).wait()
      ds[b] = fire_scatters(j + b, b, sems.at[b])
      if b >= LAG:
        drain_and_refill(b - LAG)
    for bb in range(NBUF - LAG, NBUF):
      drain_and_refill(bb)

  ds = []
  for b in range(NLOAD - NRING):
    pltpu.make_async_copy(
        nodes.at[pl.ds(row0, LCHUNK)], rows_v.at[b], seml.at[b]).wait()
    ds.extend(fire_scatters(NRING + b, b, sems.at[b]))
  for d in ds:
    d.wait()

  ball_d.wait()
  pltpu.sync_copy(ball_v, ball_out.at[pl.ds(wid * BALL_PER_TILE, BALL_PER_TILE)])

  plsc.subcore_barrier()

  pltpu.sync_copy(acc_sh.at[pl.ds(s * ACC_PER_TILE, ACC_PER_TILE)], acc_v)
  pltpu.sync_copy(acc_v, part_out.at[c, pl.ds(s * ACC_PER_TILE, ACC_PER_TILE)])
  pltpu.sync_copy(cnt_sh.at[pl.ds(s * ACC_PER_TILE, ACC_PER_TILE)], cntr_v)
  pltpu.sync_copy(cntr_v, cnt_out.at[c, pl.ds(s * ACC_PER_TILE, ACC_PER_TILE)])


def _sc_aggregate(nodes, bidx2d, bptr):
  mesh = plsc.VectorSubcoreMesh(core_axis_name="c", subcore_axis_name="s")
  return pl.kernel(
      _sc_body,
      out_type=(
          jax.ShapeDtypeStruct((NC, B, H), jnp.float32),
          jax.ShapeDtypeStruct((NC, B), jnp.float32),
          jax.ShapeDtypeStruct((B, H), jnp.float32),
      ),
      mesh=mesh,
      scratch_types=[
          pltpu.VMEM((NIDX, SCHUNK), jnp.int32),
          pltpu.VMEM((NBUF, LCHUNK, H), jnp.float32),
          pltpu.VMEM((SCHUNK,), jnp.float32),
          pltpu.VMEM((BALL_PER_TILE,), jnp.int32),
          pltpu.VMEM((BALL_PER_TILE, H), jnp.float32),
          pltpu.VMEM((ACC_PER_TILE, H), jnp.float32),
          pltpu.VMEM((ACC_PER_TILE,), jnp.float32),
          pltpu.VMEM_SHARED((B, H), jnp.float32),
          pltpu.VMEM_SHARED((B,), jnp.float32),
          pltpu.SemaphoreType.DMA,
          pltpu.SemaphoreType.DMA((NBUF,)),
          pltpu.SemaphoreType.DMA((NBUF,)),
      ],
  )(nodes, bidx2d, bptr)


def _tc_head(part_ref, cnt_ref, ball_ref, g_ref,
             bb_ref, w1_ref, b1_ref, w2_ref, b2_ref, out_ref):
  part = part_ref[...]
  seg = part[0] + part[1]
  cnt = jnp.sum(cnt_ref[...], axis=1, keepdims=True)
  ge = seg / jnp.maximum(cnt, 1.0)
  f = jnp.concatenate([ball_ref[...], ge], axis=1)
  mu = jnp.mean(f, axis=1, keepdims=True)
  d = f - mu
  var = jnp.mean(d * d, axis=1, keepdims=True)
  h = d * lax.rsqrt(var + 1e-5) * g_ref[...] + bb_ref[...]
  h = jnp.maximum(
      jnp.dot(h, w1_ref[...], preferred_element_type=jnp.float32)
      + b1_ref[...], 0.0)
  out_ref[...] = (
      jnp.dot(h, w2_ref[...], preferred_element_type=jnp.float32)
      + b2_ref[...])


def _tc_finish(part, cnt2t, ball, ln_g, ln_b, W1, b1, W2, b2):
  return pl.pallas_call(
      _tc_head,
      out_shape=jax.ShapeDtypeStruct((B, H), jnp.float32),
  )(part, cnt2t, ball, ln_g, ln_b, W1, b1, W2, b2)


@jax.jit
def _impl(node_emb, batch_ptr, batch_index, ln_g, ln_b, W1, b1, W2, b2):
  bidx = batch_index.astype(jnp.int32)
  bidx2d = bidx.reshape(NW, NIDX, SCHUNK)
  bptr = batch_ptr[:-1].astype(jnp.int32)
  part, cnt2, ball = _sc_aggregate(node_emb, bidx2d, bptr)
  return _tc_finish(part, cnt2.T, ball,
                    ln_g.reshape(1, 2 * H), ln_b.reshape(1, 2 * H),
                    W1, b1.reshape(1, H), W2, b2.reshape(1, H))


def kernel(node_emb, batch_ptr, batch_index, ln_g, ln_b, W1, b1, W2, b2):
  return _impl(node_emb, batch_ptr, batch_index, ln_g, ln_b, W1, b1, W2, b2)


# speedup vs baseline: 1.0173x; 1.0173x over previous
import jax
import jax.numpy as jnp
from jax import lax
from jax.experimental import pallas as pl
from jax.experimental.pallas import tpu as pltpu
from jax.experimental.pallas import tpu_sc as plsc

TOTAL_NODES = 131072
H = 128
B = 1024

NC = 2
NS = 16
NW = NC * NS
ROWS_PER_TILE = TOTAL_NODES // NW
SCHUNK = 128
LCHUNK = 128
SPL = LCHUNK // SCHUNK
NLOAD = ROWS_PER_TILE // LCHUNK
NBUF = 6
NRING = (NLOAD // NBUF) * NBUF
NIDX = ROWS_PER_TILE // SCHUNK
BALL_PER_TILE = B // NW
ACC_PER_TILE = B // NS


def _sc_body(nodes, bidx, bptr, part_out, cnt_out, ball_out,
             idx_v, rows_v, ones_v, bptr_v, ball_v, acc_v, cntr_v,
             acc_sh, cnt_sh, semg, semz, seml, sems):
  c = lax.axis_index("c")
  s = lax.axis_index("s")
  wid = c * NS + s
  row0 = wid * ROWS_PER_TILE

  def load(j, b, sem):
    return pltpu.async_copy(
        nodes.at[pl.ds(row0 + j * LCHUNK, LCHUNK)], rows_v.at[b], sem)

  pltpu.sync_copy(bptr.at[pl.ds(wid * BALL_PER_TILE, BALL_PER_TILE)], bptr_v)
  ball_d = pltpu.async_copy(nodes.at[bptr_v], ball_v, semg)
  for b in range(NBUF):
    load(b, b, seml.at[b])
  pltpu.sync_copy(bidx.at[wid], idx_v)

  zv = jnp.zeros((16,), jnp.float32)

  @pl.loop(0, ACC_PER_TILE)
  def _(i):
    for k in range(H // 16):
      acc_v[i, pl.ds(k * 16, 16)] = zv

  for k in range(ACC_PER_TILE // 16):
    cntr_v[pl.ds(k * 16, 16)] = zv
  z0 = pltpu.async_copy(
      acc_v, acc_sh.at[pl.ds(s * ACC_PER_TILE, ACC_PER_TILE)], semz)
  z1 = pltpu.async_copy(
      cntr_v, cnt_sh.at[pl.ds(s * ACC_PER_TILE, ACC_PER_TILE)], semz)
  z0.wait()
  z1.wait()

  for k in range(SCHUNK // 16):
    ones_v[pl.ds(k * 16, 16)] = jnp.ones((16,), jnp.float32)

  plsc.subcore_barrier()

  def fire_scatters(j, b, sem):
    return [
        pltpu.async_copy(rows_v.at[b], acc_sh.at[idx_v.at[j]], sem, add=True),
        pltpu.async_copy(ones_v, cnt_sh.at[idx_v.at[j]], sem, add=True),
    ]

  @pl.loop(0, NRING, step=NBUF)
  def _(j):
    ds = []
    for b in range(NBUF):
      pltpu.make_async_copy(
          nodes.at[pl.ds(row0, LCHUNK)], rows_v.at[b], seml.at[b]).wait()
      ds.append(fire_scatters(j + b, b, sems.at[b]))
    for b in range(NBUF):
      for d in ds[b]:
        d.wait()

      @pl.when(j + NBUF + b < NLOAD)
      def _():
        load(j + NBUF + b, b, seml.at[b])

  ds = []
  for b in range(NLOAD - NRING):
    pltpu.make_async_copy(
        nodes.at[pl.ds(row0, LCHUNK)], rows_v.at[b], seml.at[b]).wait()
    ds.extend(fire_scatters(NRING + b, b, sems.at[b]))
  for d in ds:
    d.wait()

  ball_d.wait()
  pltpu.sync_copy(ball_v, ball_out.at[pl.ds(wid * BALL_PER_TILE, BALL_PER_TILE)])

  plsc.subcore_barrier()

  r0 = pltpu.async_copy(
      acc_sh.at[pl.ds(s * ACC_PER_TILE, ACC_PER_TILE)], acc_v, semz)
  r1 = pltpu.async_copy(
      cnt_sh.at[pl.ds(s * ACC_PER_TILE, ACC_PER_TILE)], cntr_v, semz)
  r0.wait()
  r1.wait()
  r2 = pltpu.async_copy(
      acc_v, part_out.at[c, pl.ds(s * ACC_PER_TILE, ACC_PER_TILE)], semz)
  r3 = pltpu.async_copy(
      cntr_v, cnt_out.at[c, pl.ds(s * ACC_PER_TILE, ACC_PER_TILE)], semz)
  r2.wait()
  r3.wait()


def _sc_aggregate(nodes, bidx2d, bptr):
  mesh = plsc.VectorSubcoreMesh(core_axis_name="c", subcore_axis_name="s")
  return pl.kernel(
      _sc_body,
      out_type=(
          jax.ShapeDtypeStruct((NC, B, H), jnp.float32),
          jax.ShapeDtypeStruct((NC, B), jnp.float32),
          jax.ShapeDtypeStruct((B, H), jnp.float32),
      ),
      mesh=mesh,
      scratch_types=[
          pltpu.VMEM((NIDX, SCHUNK), jnp.int32),
          pltpu.VMEM((NBUF, LCHUNK, H), jnp.float32),
          pltpu.VMEM((SCHUNK,), jnp.float32),
          pltpu.VMEM((BALL_PER_TILE,), jnp.int32),
          pltpu.VMEM((BALL_PER_TILE, H), jnp.float32),
          pltpu.VMEM((ACC_PER_TILE, H), jnp.float32),
          pltpu.VMEM((ACC_PER_TILE,), jnp.float32),
          pltpu.VMEM_SHARED((B, H), jnp.float32),
          pltpu.VMEM_SHARED((B,), jnp.float32),
          pltpu.SemaphoreType.DMA,
          pltpu.SemaphoreType.DMA,
          pltpu.SemaphoreType.DMA((NBUF,)),
          pltpu.SemaphoreType.DMA((NBUF,)),
      ],
  )(nodes, bidx2d, bptr)


def _tc_head(part_ref, cnt_ref, ball_ref, g_ref,
             bb_ref, w1_ref, b1_ref, w2_ref, b2_ref, out_ref):
  part = part_ref[...]
  seg = part[0] + part[1]
  cnt = cnt_ref[...]
  c8 = (cnt[0] + cnt[1]).reshape(B // H, H)
  r8 = 1.0 / jnp.maximum(c8, 1.0)
  seg3 = seg.reshape(B // H, H, H)
  ge = (seg3 * r8[:, :, None]).reshape(B, H)
  f = jnp.concatenate([ball_ref[...], ge], axis=1)
  mu = jnp.mean(f, axis=1, keepdims=True)
  d = f - mu
  var = jnp.mean(d * d, axis=1, keepdims=True)
  h = d * lax.rsqrt(var + 1e-5) * g_ref[...] + bb_ref[...]
  h = jnp.maximum(
      jnp.dot(h, w1_ref[...], preferred_element_type=jnp.float32)
      + b1_ref[...], 0.0)
  out_ref[...] = (
      jnp.dot(h, w2_ref[...], preferred_element_type=jnp.float32)
      + b2_ref[...])


def _tc_finish(part, cnt2t, ball, ln_g, ln_b, W1, b1, W2, b2):
  return pl.pallas_call(
      _tc_head,
      out_shape=jax.ShapeDtypeStruct((B, H), jnp.float32),
  )(part, cnt2t, ball, ln_g, ln_b, W1, b1, W2, b2)


@jax.jit
def _impl(node_emb, batch_ptr, batch_index, ln_g, ln_b, W1, b1, W2, b2):
  bidx = batch_index.astype(jnp.int32)
  bidx2d = bidx.reshape(NW, NIDX, SCHUNK)
  bptr = batch_ptr[:-1].astype(jnp.int32)
  part, cnt2, ball = _sc_aggregate(node_emb, bidx2d, bptr)
  return _tc_finish(part, cnt2, ball,
                    ln_g.reshape(1, 2 * H), ln_b.reshape(1, 2 * H),
                    W1, b1.reshape(1, H), W2, b2.reshape(1, H))


def kernel(node_emb, batch_ptr, batch_index, ln_g, ln_b, W1, b1, W2, b2):
  return _impl(node_emb, batch_ptr, batch_index, ln_g, ln_b, W1, b1, W2, b2)
